# Initial kernel scaffold; baseline (speedup 1.0000x reference)
import jax
import jax.numpy as jnp
from jax.experimental import pallas as pl


def kernel(template, surf_xyz, global_feat, point_feat, W1, b1, W2, b2, Wskip, bskip, Wout, bout, Wm1, bm1, Wm2, bm2, Wm3, bm3):
    raise NotImplementedError("write your pallas kernel here")


# speedup vs baseline: 23.5594x; 23.5594x over previous
import functools

import jax
import jax.numpy as jnp
from jax.experimental import pallas as pl
from jax.experimental.pallas import tpu as pltpu

_K = 8
_EPS_D = 1e-12
_EPS_W = 1e-08
_DISP_SCALE = 0.3


def _deform_body(tmpl_ref, surf_ref, pf_ref, gvec_ref,
                 w1t_ref, w1l_ref, w1g_ref, b1_ref,
                 w2_ref, b2_ref,
                 wst_ref, wsl_ref, wsg_ref, bs_ref,
                 wot_ref, bo_ref,
                 wm1t_ref, wm1l_ref, wm1g_ref, bm1_ref,
                 wm2_ref, bm2_ref, wm3_ref, bm3_ref,
                 disp_ref, mat_ref):
    t = tmpl_ref[0]
    s3 = surf_ref[0]
    sx = s3[0:1, :]
    sy = s3[1:2, :]
    sz = s3[2:3, :]
    tx, ty, tz = t[:, 0:1], t[:, 1:2], t[:, 2:3]
    tsq = tx * tx + ty * ty + tz * tz
    ssq = sx * sx + sy * sy + sz * sz
    cross = jax.lax.dot(t.astype(jnp.bfloat16), s3.astype(jnp.bfloat16),
                        preferred_element_type=jnp.float32)
    d2 = (tsq + ssq) - 2.0 * cross

    key = d2
    m = None
    for k in range(_K):
        m = jnp.min(key, axis=1, keepdims=True)
        if k < _K - 1:
            key = jnp.where(key == m, jnp.inf, key)

    mask = d2 <= m
    dist = jnp.sqrt(jnp.maximum(d2, _EPS_D))
    w = jnp.where(mask, 1.0 / (dist + _EPS_W), 0.0)
    w = w / jnp.sum(w, axis=1, keepdims=True)

    loc = jax.lax.dot(w, pf_ref[0], preferred_element_type=jnp.float32,
                      precision=jax.lax.Precision.HIGHEST)

    def dotf(a, b):
        return jax.lax.dot(a.astype(jnp.bfloat16), b.astype(jnp.bfloat16),
                           preferred_element_type=jnp.float32)

    g = gvec_ref[0]
    gb1 = dotf(g, w1g_ref[...]) + b1_ref[...]
    gbs = dotf(g, wsg_ref[...]) + bs_ref[...]
    gbm = dotf(g, wm1g_ref[...]) + bm1_ref[...]

    h1 = jax.nn.relu(dotf(t, w1t_ref[...]) + dotf(loc, w1l_ref[...]) + gb1)
    h2 = (jax.nn.relu(dotf(h1, w2_ref[...]) + b2_ref[...])
          + dotf(t, wst_ref[...]) + dotf(loc, wsl_ref[...]) + gbs)
    disp_ref[0] = (dotf(h2, wot_ref[...]) + bo_ref[...]) * _DISP_SCALE

    m1 = jax.nn.relu(dotf(t, wm1t_ref[...]) + dotf(loc, wm1l_ref[...]) + gbm)
    m2 = jax.nn.relu(dotf(m1, wm2_ref[...]) + bm2_ref[...])
    z = dotf(m2, wm3_ref[...]) + bm3_ref[...]
    mat_ref[0] = 1.0 / (1.0 + jnp.exp(-z))


def _run(template, surf_xyz, global_feat, point_feat, params, interpret=False):
    B, T, _ = template.shape
    S = surf_xyz.shape[1]
    LD = point_feat.shape[2]
    G = global_feat.shape[1]
    RT = min(256, T)

    surf_t = surf_xyz.transpose(0, 2, 1)

    (W1, b1, W2, b2, Wskip, bskip, Wout, bout,
     Wm1, bm1, Wm2, bm2, Wm3, bm3) = params
    H = W1.shape[0]
    HM = Wm1.shape[0]
    HM2 = Wm2.shape[0]

    w1t = W1[:, :3].T
    w1l = W1[:, 3:3 + LD].T
    w1g = W1[:, 3 + LD:].T
    wst = Wskip[:, :3].T
    wsl = Wskip[:, 3:3 + LD].T
    wsg = Wskip[:, 3 + LD:].T
    wm1t = Wm1[:, :3].T
    wm1l = Wm1[:, 3:3 + LD].T
    wm1g = Wm1[:, 3 + LD:].T
    w2 = W2.T
    wot = Wout.T
    wm2 = Wm2.T
    wm3 = Wm3.T

    def row2(x):
        return x.reshape(1, -1)

    grid = (B, T // RT)

    def full_spec(shape):
        return pl.BlockSpec(shape, lambda b, i: (0,) * len(shape))

    in_specs = [
        pl.BlockSpec((1, RT, 3), lambda b, i: (b, i, 0)),
        pl.BlockSpec((1, 3, S), lambda b, i: (b, 0, 0)),
        pl.BlockSpec((1, S, LD), lambda b, i: (b, 0, 0)),
        pl.BlockSpec((1, 1, G), lambda b, i: (b, 0, 0)),
        full_spec((3, H)), full_spec((LD, H)), full_spec((G, H)), full_spec((1, H)),
        full_spec((H, H)), full_spec((1, H)),
        full_spec((3, H)), full_spec((LD, H)), full_spec((G, H)), full_spec((1, H)),
        full_spec((H, 3)), full_spec((1, 3)),
        full_spec((3, HM)), full_spec((LD, HM)), full_spec((G, HM)), full_spec((1, HM)),
        full_spec((HM, HM2)), full_spec((1, HM2)),
        full_spec((HM2, 1)), full_spec((1, 1)),
    ]
    out_specs = [
        pl.BlockSpec((1, RT, 3), lambda b, i: (b, i, 0)),
        pl.BlockSpec((1, RT, 1), lambda b, i: (b, i, 0)),
    ]
    out_shape = [
        jax.ShapeDtypeStruct((B, T, 3), jnp.float32),
        jax.ShapeDtypeStruct((B, T, 1), jnp.float32),
    ]

    disp, mat = pl.pallas_call(
        _deform_body,
        grid=grid,
        in_specs=in_specs,
        out_specs=out_specs,
        out_shape=out_shape,
        interpret=interpret,
    )(template, surf_t, point_feat, global_feat[:, None, :],
      w1t, w1l, w1g, row2(b1),
      w2, row2(b2),
      wst, wsl, wsg, row2(bskip),
      wot, row2(bout),
      wm1t, wm1l, wm1g, row2(bm1),
      wm2, row2(bm2), wm3, row2(bm3))
    return disp, mat[..., 0]


def kernel(template, surf_xyz, global_feat, point_feat, W1, b1, W2, b2,
           Wskip, bskip, Wout, bout, Wm1, bm1, Wm2, bm2, Wm3, bm3):
    params = (W1, b1, W2, b2, Wskip, bskip, Wout, bout,
              Wm1, bm1, Wm2, bm2, Wm3, bm3)
    return _run(template, surf_xyz, global_feat, point_feat, params)
